# baseline (speedup 1.0000x reference)
import functools

import jax
import jax.numpy as jnp
from jax import lax
from jax.experimental import pallas as pl
from jax.experimental.pallas import tpu as pltpu
from jax.experimental.pallas import tpu_sc as plsc

_N = 10000
_E = 160000
_C = 64

_NC = 2
_NS = 16
_NW = _NC * _NS
_EPW = _E // _NW
_CH = 128
_NFULL = _EPW // _CH
_TAIL = _EPW - _NFULL * _CH
_RPT = _N // _NS

def _sc_gather_body(x_hbm, idx_hbm, out_hbm, idx_v, rows_v, idx_t, rows_t, sem):
    c = lax.axis_index("c")
    s = lax.axis_index("s")
    wid = s * _NC + c
    base = wid * _EPW

    def chunk(j, carry):
        off = pl.multiple_of(base + j * _CH, 8)
        pltpu.sync_copy(idx_hbm.at[pl.ds(off, _CH)], idx_v)
        pltpu.async_copy(x_hbm.at[idx_v], rows_v, sem).wait()
        pltpu.sync_copy(rows_v, out_hbm.at[pl.ds(off, _CH)])
        return carry

    lax.fori_loop(0, _NFULL, chunk, 0)
    offt = pl.multiple_of(base + _NFULL * _CH, 8)
    pltpu.sync_copy(idx_hbm.at[pl.ds(offt, _TAIL)], idx_t)
    pltpu.async_copy(x_hbm.at[idx_t], rows_t, sem).wait()
    pltpu.sync_copy(rows_t, out_hbm.at[pl.ds(offt, _TAIL)])


def _sc_scatter_body(msg_hbm, dst_hbm, zero_hbm, parts_hbm, idx_v, rows_v,
                     idx_t, rows_t, agg_sh):
    c = lax.axis_index("c")
    s = lax.axis_index("s")
    wid = s * _NC + c
    pltpu.sync_copy(zero_hbm, agg_sh.at[pl.ds(s * _RPT, _RPT)])
    plsc.subcore_barrier()
    base = wid * _EPW

    def chunk(j, carry):
        off = pl.multiple_of(base + j * _CH, 8)
        pltpu.sync_copy(dst_hbm.at[pl.ds(off, _CH)], idx_v)
        pltpu.sync_copy(msg_hbm.at[pl.ds(off, _CH)], rows_v)
        pltpu.sync_copy(rows_v, agg_sh.at[idx_v], add=True)
        return carry

    lax.fori_loop(0, _NFULL, chunk, 0)
    offt = pl.multiple_of(base + _NFULL * _CH, 8)
    pltpu.sync_copy(dst_hbm.at[pl.ds(offt, _TAIL)], idx_t)
    pltpu.sync_copy(msg_hbm.at[pl.ds(offt, _TAIL)], rows_t)
    pltpu.sync_copy(rows_t, agg_sh.at[idx_t], add=True)
    plsc.subcore_barrier()
    pltpu.sync_copy(agg_sh.at[pl.ds(s * _RPT, _RPT)],
                    parts_hbm.at[pl.ds(c * _N + s * _RPT, _RPT)])


@functools.lru_cache(maxsize=None)
def _sc_kernels():
    mesh = plsc.VectorSubcoreMesh(core_axis_name="c", subcore_axis_name="s")
    params = pltpu.CompilerParams(use_tc_tiling_on_sc=False)
    gather = pl.kernel(
        _sc_gather_body,
        mesh=mesh,
        compiler_params=params,
        out_type=jax.ShapeDtypeStruct((_E, _C), jnp.float32),
        scratch_types=[
            pltpu.VMEM((_CH,), jnp.int32),
            pltpu.VMEM((_CH, _C), jnp.float32),
            pltpu.VMEM((_TAIL,), jnp.int32),
            pltpu.VMEM((_TAIL, _C), jnp.float32),
            pltpu.SemaphoreType.DMA,
        ],
    )
    scatter = pl.kernel(
        _sc_scatter_body,
        mesh=mesh,
        compiler_params=params,
        out_type=jax.ShapeDtypeStruct((2 * _N, _C), jnp.float32),
        scratch_types=[
            pltpu.VMEM((_CH,), jnp.int32),
            pltpu.VMEM((_CH, _C), jnp.float32),
            pltpu.VMEM((_TAIL,), jnp.int32),
            pltpu.VMEM((_TAIL, _C), jnp.float32),
            pltpu.VMEM_SHARED((_N, _C), jnp.float32),
        ],
    )
    return gather, scatter


_BE = 256


def _edge_body(ea_ref, xj_ref, wT_ref, nbm_ref, msg_ref):
    ea = ea_ref[...].astype(jnp.bfloat16)
    w = jnp.dot(ea, wT_ref[...], preferred_element_type=jnp.float32)
    xj = xj_ref[...]
    acc = jnp.dot(xj, nbm_ref[...], preferred_element_type=jnp.float32)
    for i in range(_C):
        acc = acc + xj[:, i:i + 1] * w[:, i * _C:(i + 1) * _C]
    msg_ref[...] = acc


def _edge_call(ea, xj, wT_bf16, nb_mat):
    return pl.pallas_call(
        _edge_body,
        grid=(_E // _BE,),
        in_specs=[
            pl.BlockSpec((_BE, _C), lambda i: (i, 0)),
            pl.BlockSpec((_BE, _C), lambda i: (i, 0)),
            pl.BlockSpec((_C, _C * _C), lambda i: (0, 0)),
            pl.BlockSpec((_C, _C), lambda i: (0, 0)),
        ],
        out_specs=pl.BlockSpec((_BE, _C), lambda i: (i, 0)),
        out_shape=jax.ShapeDtypeStruct((_E, _C), jnp.float32),
    )(ea, xj, wT_bf16, nb_mat)


_BN = 2000


def _combine_body(p_ref, x_ref, lwT_ref, b_ref, o_ref):
    s_ = (p_ref[0] + p_ref[1] + b_ref[...]
          + jnp.dot(x_ref[...], lwT_ref[...], preferred_element_type=jnp.float32))
    o_ref[...] = jnp.maximum(s_, 0.0)


def _combine_call(parts, x, lwT, b_row):
    return pl.pallas_call(
        _combine_body,
        grid=(_N // _BN,),
        in_specs=[
            pl.BlockSpec((2, _BN, _C), lambda i: (0, i, 0)),
            pl.BlockSpec((_BN, _C), lambda i: (i, 0)),
            pl.BlockSpec((_C, _C), lambda i: (0, 0)),
            pl.BlockSpec((1, _C), lambda i: (0, 0)),
        ],
        out_specs=pl.BlockSpec((_BN, _C), lambda i: (i, 0)),
        out_shape=jax.ShapeDtypeStruct((_N, _C), jnp.float32),
    )(parts, x, lwT, b_row)


def _combine_final_body(p_ref, x_ref, lwT_ref, b_ref, woT_ref, bo_ref, o_ref):
    s_ = (p_ref[0] + p_ref[1] + b_ref[...]
          + jnp.dot(x_ref[...], lwT_ref[...], preferred_element_type=jnp.float32))
    h = jnp.maximum(s_, 0.0)
    o_ref[...] = jnp.dot(h, woT_ref[...],
                         preferred_element_type=jnp.float32) + bo_ref[...]


def _combine_final_call(parts, x, lwT, b_row, woT, bo_row):
    return pl.pallas_call(
        _combine_final_body,
        grid=(_N // _BN,),
        in_specs=[
            pl.BlockSpec((2, _BN, _C), lambda i: (0, i, 0)),
            pl.BlockSpec((_BN, _C), lambda i: (i, 0)),
            pl.BlockSpec((_C, _C), lambda i: (0, 0)),
            pl.BlockSpec((1, _C), lambda i: (0, 0)),
            pl.BlockSpec((_C, _C), lambda i: (0, 0)),
            pl.BlockSpec((1, _C), lambda i: (0, 0)),
        ],
        out_specs=pl.BlockSpec((_BN, _C), lambda i: (i, 0)),
        out_shape=jax.ShapeDtypeStruct((_N, _C), jnp.float32),
    )(parts, x, lwT, b_row, woT, bo_row)


def kernel(feature, edge_index, edge_attr, nn_W0, nn_b0, lin_W0, bias0,
           nn_W1, nn_b1, lin_W1, bias1, W_out, b_out):
    src = edge_index[0]
    dst = edge_index[1]
    zero_init = jnp.zeros((_RPT, _C), jnp.float32)
    sc_gather, sc_scatter = _sc_kernels()

    def layer(x, nn_W, nn_b, lin_W, bias):
        xj = sc_gather(x, src)
        msg = _edge_call(edge_attr, xj, nn_W.T.astype(jnp.bfloat16),
                         nn_b.reshape(_C, _C))
        parts = sc_scatter(msg, dst, zero_init).reshape(2, _N, _C)
        return parts

    p0 = layer(feature, nn_W0, nn_b0, lin_W0, bias0)
    x1 = _combine_call(p0, feature, lin_W0.T, bias0.reshape(1, _C))
    p1 = layer(x1, nn_W1, nn_b1, lin_W1, bias1)
    out = _combine_final_call(p1, x1, lin_W1.T, bias1.reshape(1, _C),
                              W_out.T, b_out.reshape(1, _C))
    return out


# speedup vs baseline: 2.5948x; 2.5948x over previous
import functools

import jax
import jax.numpy as jnp
from jax import lax
from jax.experimental import pallas as pl
from jax.experimental.pallas import tpu as pltpu
from jax.experimental.pallas import tpu_sc as plsc

_N = 10000
_E = 160000
_C = 64

_NC = 2
_NS = 16
_NW = _NC * _NS
_EPW = _E // _NW
_CH = 128
_NFULL = _EPW // _CH
_TAIL = _EPW - _NFULL * _CH
_RPT = _N // _NS

def _sc_gather_body(x_hbm, idx_hbm, out_hbm, idx_v, rows_v, idx_t, rows_t, sem):
    c = lax.axis_index("c")
    s = lax.axis_index("s")
    wid = s * _NC + c
    base = wid * _EPW

    def chunk(j, carry):
        off = pl.multiple_of(base + j * _CH, 8)
        pltpu.sync_copy(idx_hbm.at[pl.ds(off, _CH)], idx_v)
        pltpu.async_copy(x_hbm.at[idx_v], rows_v, sem).wait()
        pltpu.sync_copy(rows_v, out_hbm.at[pl.ds(off, _CH)])
        return carry

    lax.fori_loop(0, _NFULL, chunk, 0)
    offt = pl.multiple_of(base + _NFULL * _CH, 8)
    pltpu.sync_copy(idx_hbm.at[pl.ds(offt, _TAIL)], idx_t)
    pltpu.async_copy(x_hbm.at[idx_t], rows_t, sem).wait()
    pltpu.sync_copy(rows_t, out_hbm.at[pl.ds(offt, _TAIL)])


def _sc_scatter_body(msg_hbm, dst_hbm, zero_hbm, parts_hbm, idx_v, rows_v,
                     idx_t, rows_t, agg_sh):
    c = lax.axis_index("c")
    s = lax.axis_index("s")
    wid = s * _NC + c
    pltpu.sync_copy(zero_hbm, agg_sh.at[pl.ds(s * _RPT, _RPT)])
    plsc.subcore_barrier()
    base = wid * _EPW

    def chunk(j, carry):
        off = pl.multiple_of(base + j * _CH, 8)
        pltpu.sync_copy(dst_hbm.at[pl.ds(off, _CH)], idx_v)
        pltpu.sync_copy(msg_hbm.at[pl.ds(off, _CH)], rows_v)
        pltpu.sync_copy(rows_v, agg_sh.at[idx_v], add=True)
        return carry

    lax.fori_loop(0, _NFULL, chunk, 0)
    offt = pl.multiple_of(base + _NFULL * _CH, 8)
    pltpu.sync_copy(dst_hbm.at[pl.ds(offt, _TAIL)], idx_t)
    pltpu.sync_copy(msg_hbm.at[pl.ds(offt, _TAIL)], rows_t)
    pltpu.sync_copy(rows_t, agg_sh.at[idx_t], add=True)
    plsc.subcore_barrier()
    pltpu.sync_copy(agg_sh.at[pl.ds(s * _RPT, _RPT)],
                    parts_hbm.at[pl.ds(c * _N + s * _RPT, _RPT)])


@functools.lru_cache(maxsize=None)
def _sc_kernels():
    mesh = plsc.VectorSubcoreMesh(core_axis_name="c", subcore_axis_name="s")
    params = pltpu.CompilerParams(use_tc_tiling_on_sc=False)
    gather = pl.kernel(
        _sc_gather_body,
        mesh=mesh,
        compiler_params=params,
        out_type=jax.ShapeDtypeStruct((_E, _C), jnp.float32),
        scratch_types=[
            pltpu.VMEM((_CH,), jnp.int32),
            pltpu.VMEM((_CH, _C), jnp.float32),
            pltpu.VMEM((_TAIL,), jnp.int32),
            pltpu.VMEM((_TAIL, _C), jnp.float32),
            pltpu.SemaphoreType.DMA,
        ],
    )
    scatter = pl.kernel(
        _sc_scatter_body,
        mesh=mesh,
        compiler_params=params,
        out_type=jax.ShapeDtypeStruct((2 * _N, _C), jnp.float32),
        scratch_types=[
            pltpu.VMEM((_CH,), jnp.int32),
            pltpu.VMEM((_CH, _C), jnp.float32),
            pltpu.VMEM((_TAIL,), jnp.int32),
            pltpu.VMEM((_TAIL, _C), jnp.float32),
            pltpu.VMEM_SHARED((_N, _C), jnp.float32),
        ],
    )
    return gather, scatter


_BE = 256


def _edge_body(ea_ref, xj_ref, T2T_ref, nbmT_ref, sel_ref, msg_ref):
    eaT = ea_ref[...].T
    xjT = xj_ref[...].T.astype(jnp.bfloat16)
    ZT = jnp.dot(T2T_ref[...], xjT,
                 preferred_element_type=jnp.float32)
    eaT_t = jnp.tile(eaT, (_C, 1))
    P = (eaT_t * ZT).astype(jnp.bfloat16)
    msgT = jnp.dot(sel_ref[...], P, preferred_element_type=jnp.float32)
    msgT = msgT + jnp.dot(nbmT_ref[...], xjT,
                          preferred_element_type=jnp.float32)
    msg_ref[...] = msgT.T


def _edge_call(ea, xj, T2T_bf16, nbmT, sel_bf16):
    return pl.pallas_call(
        _edge_body,
        grid=(_E // _BE,),
        in_specs=[
            pl.BlockSpec((_BE, _C), lambda i: (i, 0)),
            pl.BlockSpec((_BE, _C), lambda i: (i, 0)),
            pl.BlockSpec((_C * _C, _C), lambda i: (0, 0)),
            pl.BlockSpec((_C, _C), lambda i: (0, 0)),
            pl.BlockSpec((_C, _C * _C), lambda i: (0, 0)),
        ],
        out_specs=pl.BlockSpec((_BE, _C), lambda i: (i, 0)),
        out_shape=jax.ShapeDtypeStruct((_E, _C), jnp.float32),
    )(ea, xj, T2T_bf16, nbmT, sel_bf16)


_BN = 2000


def _combine_body(p_ref, x_ref, lwT_ref, b_ref, o_ref):
    s_ = (p_ref[0] + p_ref[1] + b_ref[...]
          + jnp.dot(x_ref[...], lwT_ref[...], preferred_element_type=jnp.float32))
    o_ref[...] = jnp.maximum(s_, 0.0)


def _combine_call(parts, x, lwT, b_row):
    return pl.pallas_call(
        _combine_body,
        grid=(_N // _BN,),
        in_specs=[
            pl.BlockSpec((2, _BN, _C), lambda i: (0, i, 0)),
            pl.BlockSpec((_BN, _C), lambda i: (i, 0)),
            pl.BlockSpec((_C, _C), lambda i: (0, 0)),
            pl.BlockSpec((1, _C), lambda i: (0, 0)),
        ],
        out_specs=pl.BlockSpec((_BN, _C), lambda i: (i, 0)),
        out_shape=jax.ShapeDtypeStruct((_N, _C), jnp.float32),
    )(parts, x, lwT, b_row)


def _combine_final_body(p_ref, x_ref, lwT_ref, b_ref, woT_ref, bo_ref, o_ref):
    s_ = (p_ref[0] + p_ref[1] + b_ref[...]
          + jnp.dot(x_ref[...], lwT_ref[...], preferred_element_type=jnp.float32))
    h = jnp.maximum(s_, 0.0)
    o_ref[...] = jnp.dot(h, woT_ref[...],
                         preferred_element_type=jnp.float32) + bo_ref[...]


def _combine_final_call(parts, x, lwT, b_row, woT, bo_row):
    return pl.pallas_call(
        _combine_final_body,
        grid=(_N // _BN,),
        in_specs=[
            pl.BlockSpec((2, _BN, _C), lambda i: (0, i, 0)),
            pl.BlockSpec((_BN, _C), lambda i: (i, 0)),
            pl.BlockSpec((_C, _C), lambda i: (0, 0)),
            pl.BlockSpec((1, _C), lambda i: (0, 0)),
            pl.BlockSpec((_C, _C), lambda i: (0, 0)),
            pl.BlockSpec((1, _C), lambda i: (0, 0)),
        ],
        out_specs=pl.BlockSpec((_BN, _C), lambda i: (i, 0)),
        out_shape=jax.ShapeDtypeStruct((_N, _C), jnp.float32),
    )(parts, x, lwT, b_row, woT, bo_row)


def kernel(feature, edge_index, edge_attr, nn_W0, nn_b0, lin_W0, bias0,
           nn_W1, nn_b1, lin_W1, bias1, W_out, b_out):
    src = edge_index[0]
    dst = edge_index[1]
    zero_init = jnp.zeros((_RPT, _C), jnp.float32)
    sc_gather, sc_scatter = _sc_kernels()

    sel = jnp.eye(_C, dtype=jnp.bfloat16)[:, :, None]
    sel = jnp.broadcast_to(sel, (_C, _C, _C)).reshape(_C, _C * _C)

    def layer(x, nn_W, nn_b, lin_W, bias):
        xj = sc_gather(x, src)
        T2T = nn_W.reshape(_C, _C, _C).transpose(1, 2, 0).reshape(
            _C * _C, _C).astype(jnp.bfloat16)
        nbmT = nn_b.reshape(_C, _C).T.astype(jnp.bfloat16)
        msg = _edge_call(edge_attr, xj, T2T, nbmT, sel)
        parts = sc_scatter(msg, dst, zero_init).reshape(2, _N, _C)
        return parts

    p0 = layer(feature, nn_W0, nn_b0, lin_W0, bias0)
    x1 = _combine_call(p0, feature, lin_W0.T, bias0.reshape(1, _C))
    p1 = layer(x1, nn_W1, nn_b1, lin_W1, bias1)
    out = _combine_final_call(p1, x1, lin_W1.T, bias1.reshape(1, _C),
                              W_out.T, b_out.reshape(1, _C))
    return out


# speedup vs baseline: 3.7531x; 1.4464x over previous
import functools

import jax
import jax.numpy as jnp
from jax import lax
from jax.experimental import pallas as pl
from jax.experimental.pallas import tpu as pltpu
from jax.experimental.pallas import tpu_sc as plsc

_N = 10000
_E = 160000
_C = 64

_NC = 2
_NS = 16
_NW = _NC * _NS
_EPW = _E // _NW
_CH = 128
_NFULL = _EPW // _CH
_TAIL = _EPW - _NFULL * _CH
_RPT = _N // _NS

def _sc_gather_body(x_hbm, idx_hbm, out_hbm, idx_v, rows_v, idx_t, rows_t, sem):
    c = lax.axis_index("c")
    s = lax.axis_index("s")
    wid = s * _NC + c
    base = wid * _EPW

    def chunk(j, carry):
        off = pl.multiple_of(base + j * _CH, 8)
        pltpu.sync_copy(idx_hbm.at[pl.ds(off, _CH)], idx_v)
        pltpu.async_copy(x_hbm.at[idx_v], rows_v, sem).wait()
        pltpu.sync_copy(rows_v, out_hbm.at[pl.ds(off, _CH)])
        return carry

    lax.fori_loop(0, _NFULL, chunk, 0)
    offt = pl.multiple_of(base + _NFULL * _CH, 8)
    pltpu.sync_copy(idx_hbm.at[pl.ds(offt, _TAIL)], idx_t)
    pltpu.async_copy(x_hbm.at[idx_t], rows_t, sem).wait()
    pltpu.sync_copy(rows_t, out_hbm.at[pl.ds(offt, _TAIL)])


def _sc_scatter_body(msg_hbm, dst_hbm, zero_hbm, parts_hbm, idx_v, rows_v,
                     idx_t, rows_t, agg_sh):
    c = lax.axis_index("c")
    s = lax.axis_index("s")
    wid = s * _NC + c
    pltpu.sync_copy(zero_hbm, agg_sh.at[pl.ds(s * _RPT, _RPT)])
    plsc.subcore_barrier()
    base = wid * _EPW

    def chunk(j, carry):
        off = pl.multiple_of(base + j * _CH, 8)
        pltpu.sync_copy(dst_hbm.at[pl.ds(off, _CH)], idx_v)
        pltpu.sync_copy(msg_hbm.at[pl.ds(off, _CH)], rows_v)
        pltpu.sync_copy(rows_v, agg_sh.at[idx_v], add=True)
        return carry

    lax.fori_loop(0, _NFULL, chunk, 0)
    offt = pl.multiple_of(base + _NFULL * _CH, 8)
    pltpu.sync_copy(dst_hbm.at[pl.ds(offt, _TAIL)], idx_t)
    pltpu.sync_copy(msg_hbm.at[pl.ds(offt, _TAIL)], rows_t)
    pltpu.sync_copy(rows_t, agg_sh.at[idx_t], add=True)
    plsc.subcore_barrier()
    pltpu.sync_copy(agg_sh.at[pl.ds(s * _RPT, _RPT)],
                    parts_hbm.at[pl.ds(c * _N + s * _RPT, _RPT)])


@functools.lru_cache(maxsize=None)
def _sc_kernels():
    mesh = plsc.VectorSubcoreMesh(core_axis_name="c", subcore_axis_name="s")
    params = pltpu.CompilerParams(use_tc_tiling_on_sc=False)
    gather = pl.kernel(
        _sc_gather_body,
        mesh=mesh,
        compiler_params=params,
        out_type=jax.ShapeDtypeStruct((_E, _C), jnp.float32),
        scratch_types=[
            pltpu.VMEM((_CH,), jnp.int32),
            pltpu.VMEM((_CH, _C), jnp.float32),
            pltpu.VMEM((_TAIL,), jnp.int32),
            pltpu.VMEM((_TAIL, _C), jnp.float32),
            pltpu.SemaphoreType.DMA,
        ],
    )
    scatter = pl.kernel(
        _sc_scatter_body,
        mesh=mesh,
        compiler_params=params,
        out_type=jax.ShapeDtypeStruct((2 * _N, _C), jnp.float32),
        scratch_types=[
            pltpu.VMEM((_CH,), jnp.int32),
            pltpu.VMEM((_CH, _C), jnp.float32),
            pltpu.VMEM((_TAIL,), jnp.int32),
            pltpu.VMEM((_TAIL, _C), jnp.float32),
            pltpu.VMEM_SHARED((_N, _C), jnp.float32),
        ],
    )
    return gather, scatter


_BE = 256


def _edge_body(ea_ref, xj_ref, W2T_ref, msg_ref):
    eaT = ea_ref[...].T
    xjT = xj_ref[...].T
    rep = jnp.broadcast_to(xjT[:, None, :],
                           (_C, _C, _BE)).reshape(_C * _C, _BE)
    eaT_t = jnp.tile(eaT, (_C, 1))
    u = (rep * eaT_t).astype(jnp.bfloat16)
    u_full = jnp.concatenate([u, xjT.astype(jnp.bfloat16)], axis=0)
    msgT = jnp.dot(W2T_ref[...], u_full,
                   preferred_element_type=jnp.float32)
    msg_ref[...] = msgT.T


def _edge_call(ea, xj, W2T_bf16):
    return pl.pallas_call(
        _edge_body,
        grid=(_E // _BE,),
        in_specs=[
            pl.BlockSpec((_BE, _C), lambda i: (i, 0)),
            pl.BlockSpec((_BE, _C), lambda i: (i, 0)),
            pl.BlockSpec((_C, _C * _C + _C), lambda i: (0, 0)),
        ],
        out_specs=pl.BlockSpec((_BE, _C), lambda i: (i, 0)),
        out_shape=jax.ShapeDtypeStruct((_E, _C), jnp.float32),
    )(ea, xj, W2T_bf16)


_BN = 2000


def _combine_body(p_ref, x_ref, lwT_ref, b_ref, o_ref):
    s_ = (p_ref[0] + p_ref[1] + b_ref[...]
          + jnp.dot(x_ref[...], lwT_ref[...], preferred_element_type=jnp.float32))
    o_ref[...] = jnp.maximum(s_, 0.0)


def _combine_call(parts, x, lwT, b_row):
    return pl.pallas_call(
        _combine_body,
        grid=(_N // _BN,),
        in_specs=[
            pl.BlockSpec((2, _BN, _C), lambda i: (0, i, 0)),
            pl.BlockSpec((_BN, _C), lambda i: (i, 0)),
            pl.BlockSpec((_C, _C), lambda i: (0, 0)),
            pl.BlockSpec((1, _C), lambda i: (0, 0)),
        ],
        out_specs=pl.BlockSpec((_BN, _C), lambda i: (i, 0)),
        out_shape=jax.ShapeDtypeStruct((_N, _C), jnp.float32),
    )(parts, x, lwT, b_row)


def _combine_final_body(p_ref, x_ref, lwT_ref, b_ref, woT_ref, bo_ref, o_ref):
    s_ = (p_ref[0] + p_ref[1] + b_ref[...]
          + jnp.dot(x_ref[...], lwT_ref[...], preferred_element_type=jnp.float32))
    h = jnp.maximum(s_, 0.0)
    o_ref[...] = jnp.dot(h, woT_ref[...],
                         preferred_element_type=jnp.float32) + bo_ref[...]


def _combine_final_call(parts, x, lwT, b_row, woT, bo_row):
    return pl.pallas_call(
        _combine_final_body,
        grid=(_N // _BN,),
        in_specs=[
            pl.BlockSpec((2, _BN, _C), lambda i: (0, i, 0)),
            pl.BlockSpec((_BN, _C), lambda i: (i, 0)),
            pl.BlockSpec((_C, _C), lambda i: (0, 0)),
            pl.BlockSpec((1, _C), lambda i: (0, 0)),
            pl.BlockSpec((_C, _C), lambda i: (0, 0)),
            pl.BlockSpec((1, _C), lambda i: (0, 0)),
        ],
        out_specs=pl.BlockSpec((_BN, _C), lambda i: (i, 0)),
        out_shape=jax.ShapeDtypeStruct((_N, _C), jnp.float32),
    )(parts, x, lwT, b_row, woT, bo_row)


def kernel(feature, edge_index, edge_attr, nn_W0, nn_b0, lin_W0, bias0,
           nn_W1, nn_b1, lin_W1, bias1, W_out, b_out):
    src = edge_index[0]
    dst = edge_index[1]
    zero_init = jnp.zeros((_RPT, _C), jnp.float32)
    sc_gather, sc_scatter = _sc_kernels()

    def layer(x, nn_W, nn_b, lin_W, bias):
        xj = sc_gather(x, src)
        W2T = nn_W.reshape(_C, _C, _C).transpose(1, 0, 2).reshape(_C, _C * _C)
        nbmT = nn_b.reshape(_C, _C).T
        W2T = jnp.concatenate([W2T, nbmT], axis=1).astype(jnp.bfloat16)
        msg = _edge_call(edge_attr, xj, W2T)
        parts = sc_scatter(msg, dst, zero_init).reshape(2, _N, _C)
        return parts

    p0 = layer(feature, nn_W0, nn_b0, lin_W0, bias0)
    x1 = _combine_call(p0, feature, lin_W0.T, bias0.reshape(1, _C))
    p1 = layer(x1, nn_W1, nn_b1, lin_W1, bias1)
    out = _combine_final_call(p1, x1, lin_W1.T, bias1.reshape(1, _C),
                              W_out.T, b_out.reshape(1, _C))
    return out


# speedup vs baseline: 5.4391x; 1.4492x over previous
import functools

import jax
import jax.numpy as jnp
from jax import lax
from jax.experimental import pallas as pl
from jax.experimental.pallas import tpu as pltpu
from jax.experimental.pallas import tpu_sc as plsc

_N = 10000
_E = 160000
_C = 64

_NC = 2
_NS = 16
_NW = _NC * _NS
_EPW = _E // _NW
_CH = 128
_NFULL = _EPW // _CH
_TAIL = _EPW - _NFULL * _CH
_RPT = _N // _NS

def _sc_gather_body(x_hbm, idx_hbm, out_hbm, idx_v, rows_v, idx_t, rows_t, sem):
    c = lax.axis_index("c")
    s = lax.axis_index("s")
    wid = s * _NC + c
    base = wid * _EPW

    def chunk(j, carry):
        off = pl.multiple_of(base + j * _CH, 8)
        pltpu.sync_copy(idx_hbm.at[pl.ds(off, _CH)], idx_v)
        pltpu.async_copy(x_hbm.at[idx_v], rows_v, sem).wait()
        pltpu.sync_copy(rows_v, out_hbm.at[pl.ds(off, _CH)])
        return carry

    lax.fori_loop(0, _NFULL, chunk, 0)
    offt = pl.multiple_of(base + _NFULL * _CH, 8)
    pltpu.sync_copy(idx_hbm.at[pl.ds(offt, _TAIL)], idx_t)
    pltpu.async_copy(x_hbm.at[idx_t], rows_t, sem).wait()
    pltpu.sync_copy(rows_t, out_hbm.at[pl.ds(offt, _TAIL)])


def _sc_scatter_body(msg_hbm, dst_hbm, zero_hbm, parts_hbm, idx_v, rows_v,
                     idx_t, rows_t, agg_sh):
    c = lax.axis_index("c")
    s = lax.axis_index("s")
    wid = s * _NC + c
    pltpu.sync_copy(zero_hbm, agg_sh.at[pl.ds(s * _RPT, _RPT)])
    plsc.subcore_barrier()
    base = wid * _EPW

    def chunk(j, carry):
        off = pl.multiple_of(base + j * _CH, 8)
        pltpu.sync_copy(dst_hbm.at[pl.ds(off, _CH)], idx_v)
        pltpu.sync_copy(msg_hbm.at[pl.ds(off, _CH)], rows_v)
        pltpu.sync_copy(rows_v, agg_sh.at[idx_v], add=True)
        return carry

    lax.fori_loop(0, _NFULL, chunk, 0)
    offt = pl.multiple_of(base + _NFULL * _CH, 8)
    pltpu.sync_copy(dst_hbm.at[pl.ds(offt, _TAIL)], idx_t)
    pltpu.sync_copy(msg_hbm.at[pl.ds(offt, _TAIL)], rows_t)
    pltpu.sync_copy(rows_t, agg_sh.at[idx_t], add=True)
    plsc.subcore_barrier()
    pltpu.sync_copy(agg_sh.at[pl.ds(s * _RPT, _RPT)],
                    parts_hbm.at[pl.ds(c * _N + s * _RPT, _RPT)])


@functools.lru_cache(maxsize=None)
def _sc_kernels():
    mesh = plsc.VectorSubcoreMesh(core_axis_name="c", subcore_axis_name="s")
    params = pltpu.CompilerParams(use_tc_tiling_on_sc=False)
    gather = pl.kernel(
        _sc_gather_body,
        mesh=mesh,
        compiler_params=params,
        out_type=jax.ShapeDtypeStruct((_E, _C), jnp.float32),
        scratch_types=[
            pltpu.VMEM((_CH,), jnp.int32),
            pltpu.VMEM((_CH, _C), jnp.float32),
            pltpu.VMEM((_TAIL,), jnp.int32),
            pltpu.VMEM((_TAIL, _C), jnp.float32),
            pltpu.SemaphoreType.DMA,
        ],
    )
    scatter = pl.kernel(
        _sc_scatter_body,
        mesh=mesh,
        compiler_params=params,
        out_type=jax.ShapeDtypeStruct((2 * _N, _C), jnp.float32),
        scratch_types=[
            pltpu.VMEM((_CH,), jnp.int32),
            pltpu.VMEM((_CH, _C), jnp.float32),
            pltpu.VMEM((_TAIL,), jnp.int32),
            pltpu.VMEM((_TAIL, _C), jnp.float32),
            pltpu.VMEM_SHARED((_N, _C), jnp.float32),
        ],
    )
    return gather, scatter


_BE = 2048
_BH = 256


def _edge_half(ea, xj, W2T):
    eaT = ea.T
    xjT = xj.T
    rep = jnp.broadcast_to(xjT[:, None, :],
                           (_C, _C, _BH)).reshape(_C * _C, _BH)
    eaT_t = jnp.tile(eaT, (_C, 1))
    u = (rep * eaT_t).astype(jnp.bfloat16)
    u_full = jnp.concatenate([u, xjT.astype(jnp.bfloat16)], axis=0)
    msgT = jnp.dot(W2T, u_full,
                   preferred_element_type=jnp.float32)
    return msgT.T


def _edge_body(ea_ref, xj_ref, W2T_ref, msg_ref):
    W2T = W2T_ref[...]
    for h in range(_BE // _BH):
        sl = pl.ds(h * _BH, _BH)
        msg_ref[sl, :] = _edge_half(ea_ref[sl, :], xj_ref[sl, :], W2T)


def _edge_call(ea, xj, W2T_bf16):
    return pl.pallas_call(
        _edge_body,
        grid=(pl.cdiv(_E, _BE),),
        in_specs=[
            pl.BlockSpec((_BE, _C), lambda i: (i, 0)),
            pl.BlockSpec((_BE, _C), lambda i: (i, 0)),
            pl.BlockSpec((_C, _C * _C + _C), lambda i: (0, 0)),
        ],
        out_specs=pl.BlockSpec((_BE, _C), lambda i: (i, 0)),
        out_shape=jax.ShapeDtypeStruct((_E, _C), jnp.float32),
    )(ea, xj, W2T_bf16)


_BN = 2000


def _combine_body(p_ref, x_ref, lwT_ref, b_ref, o_ref):
    s_ = (p_ref[0] + p_ref[1] + b_ref[...]
          + jnp.dot(x_ref[...], lwT_ref[...], preferred_element_type=jnp.float32))
    o_ref[...] = jnp.maximum(s_, 0.0)


def _combine_call(parts, x, lwT, b_row):
    return pl.pallas_call(
        _combine_body,
        grid=(_N // _BN,),
        in_specs=[
            pl.BlockSpec((2, _BN, _C), lambda i: (0, i, 0)),
            pl.BlockSpec((_BN, _C), lambda i: (i, 0)),
            pl.BlockSpec((_C, _C), lambda i: (0, 0)),
            pl.BlockSpec((1, _C), lambda i: (0, 0)),
        ],
        out_specs=pl.BlockSpec((_BN, _C), lambda i: (i, 0)),
        out_shape=jax.ShapeDtypeStruct((_N, _C), jnp.float32),
    )(parts, x, lwT, b_row)


def _combine_final_body(p_ref, x_ref, lwT_ref, b_ref, woT_ref, bo_ref, o_ref):
    s_ = (p_ref[0] + p_ref[1] + b_ref[...]
          + jnp.dot(x_ref[...], lwT_ref[...], preferred_element_type=jnp.float32))
    h = jnp.maximum(s_, 0.0)
    o_ref[...] = jnp.dot(h, woT_ref[...],
                         preferred_element_type=jnp.float32) + bo_ref[...]


def _combine_final_call(parts, x, lwT, b_row, woT, bo_row):
    return pl.pallas_call(
        _combine_final_body,
        grid=(_N // _BN,),
        in_specs=[
            pl.BlockSpec((2, _BN, _C), lambda i: (0, i, 0)),
            pl.BlockSpec((_BN, _C), lambda i: (i, 0)),
            pl.BlockSpec((_C, _C), lambda i: (0, 0)),
            pl.BlockSpec((1, _C), lambda i: (0, 0)),
            pl.BlockSpec((_C, _C), lambda i: (0, 0)),
            pl.BlockSpec((1, _C), lambda i: (0, 0)),
        ],
        out_specs=pl.BlockSpec((_BN, _C), lambda i: (i, 0)),
        out_shape=jax.ShapeDtypeStruct((_N, _C), jnp.float32),
    )(parts, x, lwT, b_row, woT, bo_row)


def kernel(feature, edge_index, edge_attr, nn_W0, nn_b0, lin_W0, bias0,
           nn_W1, nn_b1, lin_W1, bias1, W_out, b_out):
    src = edge_index[0]
    dst = edge_index[1]
    zero_init = jnp.zeros((_RPT, _C), jnp.float32)
    sc_gather, sc_scatter = _sc_kernels()

    def layer(x, nn_W, nn_b, lin_W, bias):
        xj = sc_gather(x, src)
        W2T = nn_W.reshape(_C, _C, _C).transpose(1, 0, 2).reshape(_C, _C * _C)
        nbmT = nn_b.reshape(_C, _C).T
        W2T = jnp.concatenate([W2T, nbmT], axis=1).astype(jnp.bfloat16)
        msg = _edge_call(edge_attr, xj, W2T)
        parts = sc_scatter(msg, dst, zero_init).reshape(2, _N, _C)
        return parts

    p0 = layer(feature, nn_W0, nn_b0, lin_W0, bias0)
    x1 = _combine_call(p0, feature, lin_W0.T, bias0.reshape(1, _C))
    p1 = layer(x1, nn_W1, nn_b1, lin_W1, bias1)
    out = _combine_final_call(p1, x1, lin_W1.T, bias1.reshape(1, _C),
                              W_out.T, b_out.reshape(1, _C))
    return out
